# baseline (speedup 1.0000x reference)
import jax
import jax.numpy as jnp
from jax import lax
from jax.experimental import pallas as pl
from jax.experimental.pallas import tpu as pltpu
from jax.experimental.pallas import tpu_sc as plsc

MIN_KEPT_PER_BATCH = 100000

B = 16
H = 512
W = 512
N = B * H * W
NT = 16
KROW = 256
ROWS_PER_CHUNK = 16
CHUNK_W = ROWS_PER_CHUNK * W
NCHUNK = KROW // ROWS_PER_CHUNK
NBUCKET = 32768
HROW = NBUCKET // 128
SLICE_ROWS = HROW // NT


def _tc_keys_body(l_ref, g_ref, o_ref):
    x = l_ref[...]
    t = g_ref[...]
    loss = jnp.maximum(x, 0.0) - x * t + jnp.log1p(jnp.exp(-jnp.abs(x)))
    k = jnp.right_shift(lax.bitcast_convert_type(loss, jnp.int32), 16)
    a = k[:, : H // 2, :]
    b = k[:, H // 2:, :]
    o_ref[...] = a | lax.shift_left(b, 16)


def _tc_keys(logits3, gts3):
    return pl.pallas_call(
        _tc_keys_body,
        grid=(4,),
        in_specs=[
            pl.BlockSpec((B // 4, H, W), lambda i: (i, 0, 0)),
            pl.BlockSpec((B // 4, H, W), lambda i: (i, 0, 0)),
        ],
        out_specs=pl.BlockSpec((B // 4, KROW, W), lambda i: (i, 0, 0)),
        out_shape=jax.ShapeDtypeStruct((B, KROW, W), jnp.int32),
    )(logits3, gts3)


def _extract(vec, i):
    lanes = lax.iota(jnp.int32, 16)
    return jnp.sum(jnp.where(lanes == i, vec, 0.0))


def _mid_value(bucket_base, lanes):
    key = lax.shift_left(bucket_base + lanes, 16) | 0x8000
    return plsc.bitcast(key, jnp.float32)


def _sc_select(keys3, kept):
    kept_f = float(kept)

    def body(keys_hbm, out_hbm, bufa, bufb, cnt, cnt2, scn_c, idx,
             rbuf, tbuf, sema, semb, merged_c, totals):
        s = lax.axis_index("s")
        wid = s
        ones = jnp.full((16,), 1.0, jnp.float32)
        zeros16 = jnp.zeros((16,), jnp.float32)
        lanes = lax.iota(jnp.int32, 16)

        def chunk_src(ci):
            return keys_hbm.at[s, pl.ds(ci * ROWS_PER_CHUNK, ROWS_PER_CHUNK), :]

        def start(ci, buf, sem):
            pltpu.make_async_copy(chunk_src(ci), buf, sem).start()

        def wait(ci, buf, sem):
            pltpu.make_async_copy(chunk_src(ci), buf, sem).wait()

        @plsc.parallel_loop(0, HROW, unroll=8)
        def _(r):
            for c in range(8):
                cnt[r, pl.ds(c * 16, 16)] = zeros16
                cnt2[r, pl.ds(c * 16, 16)] = zeros16

        def fill_idx(h, _):
            def fv(i, _):
                idx[h, pl.ds(i * 16, 16)] = (
                    lax.iota(jnp.int32, 16) + h * 128 + i * 16)
                return 0
            lax.fori_loop(0, 8, fv, 0)
            return 0
        lax.fori_loop(0, HROW // 128, fill_idx, 0)

        @pl.when(wid == 0)
        def _():
            pltpu.sync_copy(cnt, merged_c)

        def process(buf):
            @plsc.parallel_loop(0, CHUNK_W // 16, unroll=8)
            def _(i):
                r = jnp.right_shift(i, 5)
                c = i & 31
                v = buf[r, pl.ds(c * 16, 16)]
                lo = v & 0xFFFF
                hi = lax.shift_right_logical(v, 16)
                plsc.addupdate_scatter(
                    cnt, [jnp.right_shift(lo, 7), lo & 127], ones)
                plsc.addupdate_scatter(
                    cnt2, [jnp.right_shift(hi, 7), hi & 127], ones)

        start(0, bufa, sema)

        def pair(p, _):
            c0 = 2 * p
            wait(c0, bufa, sema)
            start(c0 + 1, bufb, semb)
            process(bufa)
            wait(c0 + 1, bufb, semb)

            @pl.when(c0 + 2 < NCHUNK)
            def _():
                start(c0 + 2, bufa, sema)

            process(bufb)
            return 0

        lax.fori_loop(0, NCHUNK // 2, pair, 0)

        plsc.subcore_barrier()

        for h in range(HROW // 128):
            pltpu.sync_copy(cnt.at[pl.ds(h * 128, 128), :],
                            merged_c.at[idx.at[h]], add=True)
            pltpu.sync_copy(cnt2.at[pl.ds(h * 128, 128), :],
                            merged_c.at[idx.at[h]], add=True)
        plsc.subcore_barrier()

        pltpu.sync_copy(merged_c.at[pl.ds(s * SLICE_ROWS, SLICE_ROWS), :],
                        scn_c)

        def tot(i, carry):
            vc, vs = carry
            r = jnp.right_shift(i, 3)
            c = i & 7
            cv = scn_c[r, pl.ds(c * 16, 16)]
            base = (s * SLICE_ROWS + r) * 128 + c * 16
            return (vc + cv, vs + cv * _mid_value(base, lanes))
        vc, vs = lax.fori_loop(0, SLICE_ROWS * 8, tot, (zeros16, zeros16))
        tc_ = jnp.sum(vc)
        ts_ = jnp.sum(vs)
        rbuf[...] = (jnp.where(lanes == 0, tc_, 0.0)
                     + jnp.where(lanes == 1, ts_, 0.0))
        pltpu.sync_copy(rbuf, totals.at[s])
        plsc.subcore_barrier()

        @pl.when(wid == 0)
        def _():
            pltpu.sync_copy(totals, tbuf)

            def coarse(t, carry):
                cum_c, cum_s, sstar, base_c, base_s = carry
                tt = NT - 1 - t
                rv = tbuf[tt]
                tcv = rv[0]
                tsv = rv[1]
                hit = jnp.logical_and(cum_c + tcv >= kept_f, sstar < 0)
                sstar = jnp.where(hit, tt, sstar)
                base_c = jnp.where(hit, cum_c, base_c)
                base_s = jnp.where(hit, cum_s, base_s)
                return (cum_c + tcv, cum_s + tsv, sstar, base_c, base_s)

            _, _, sstar, base_c, base_s = lax.fori_loop(
                0, NT, coarse,
                (0.0, 0.0, jnp.int32(-1), 0.0, 0.0))

            pltpu.sync_copy(
                merged_c.at[pl.ds(sstar * SLICE_ROWS, SLICE_ROWS), :], scn_c)

            def fine(j, carry):
                (cum_c, cum_s, found, cnt_ab, sum_ab, mstar) = carry
                r = SLICE_ROWS - 1 - jnp.right_shift(j, 3)
                cj = 7 - (j & 7)
                vcv = scn_c[r, pl.ds(cj * 16, 16)]
                base = (sstar * SLICE_ROWS + r) * 128 + cj * 16
                midv = _mid_value(base, lanes)
                vsv = vcv * midv
                rc = lax.rev(vcv, (0,))
                rs = lax.rev(vsv, (0,))
                rm = lax.rev(midv, (0,))
                cc = plsc.cumsum(rc)
                cs = plsc.cumsum(rs)
                sfx = cum_c + cc
                msk = sfx >= kept_f
                ntrue = plsc.all_reduce_population_count(msk)[0]
                i0 = 16 - ntrue
                cc_i = _extract(cc, i0)
                cs_i = _extract(cs, i0)
                rc_i = _extract(rc, i0)
                rs_i = _extract(rs, i0)
                rm_i = _extract(rm, i0)
                use = jnp.logical_and(ntrue > 0, jnp.logical_not(found))
                cnt_ab = jnp.where(use, cum_c + cc_i - rc_i, cnt_ab)
                sum_ab = jnp.where(use, cum_s + cs_i - rs_i, sum_ab)
                mstar = jnp.where(use, rm_i, mstar)
                found = jnp.logical_or(found, ntrue > 0)
                return (cum_c + cc[15], cum_s + cs[15], found,
                        cnt_ab, sum_ab, mstar)

            (_, _, _, cnt_ab, sum_ab, mstar) = lax.fori_loop(
                0, SLICE_ROWS * 8, fine,
                (base_c, base_s, jnp.bool_(False), 0.0, 0.0, 0.0))

            resid = kept_f - cnt_ab
            ans = jnp.full((16,), (sum_ab + resid * mstar) * (1.0 / kept_f),
                           jnp.float32)
            rbuf[...] = ans
            pltpu.sync_copy(rbuf, out_hbm)

    mesh = plsc.VectorSubcoreMesh(
        core_axis_name="c", subcore_axis_name="s", num_cores=1)
    f = pl.kernel(
        body,
        out_type=jax.ShapeDtypeStruct((16,), jnp.float32),
        mesh=mesh,
        compiler_params=pltpu.CompilerParams(needs_layout_passes=False),
        scratch_types=[
            pltpu.VMEM((ROWS_PER_CHUNK, W), jnp.int32),
            pltpu.VMEM((ROWS_PER_CHUNK, W), jnp.int32),
            pltpu.VMEM((HROW, 128), jnp.float32),
            pltpu.VMEM((HROW, 128), jnp.float32),
            pltpu.VMEM((SLICE_ROWS, 128), jnp.float32),
            pltpu.VMEM((HROW // 128, 128), jnp.int32),
            pltpu.VMEM((16,), jnp.float32),
            pltpu.VMEM((NT, 16), jnp.float32),
            pltpu.SemaphoreType.DMA,
            pltpu.SemaphoreType.DMA,
            pltpu.VMEM_SHARED((HROW, 128), jnp.float32),
            pltpu.VMEM_SHARED((NT, 16), jnp.float32),
        ],
    )
    return f(keys3)


@jax.jit
def kernel(logits, gts):
    kept = MIN_KEPT_PER_BATCH * gts.shape[0]
    l3 = logits.reshape(B, H, W)
    g3 = gts.reshape(B, H, W)
    keys = _tc_keys(l3, g3)
    out = _sc_select(keys, kept)
    return out[0]


# speedup vs baseline: 1.0368x; 1.0368x over previous
import jax
import jax.numpy as jnp
from jax import lax
from jax.experimental import pallas as pl
from jax.experimental.pallas import tpu as pltpu
from jax.experimental.pallas import tpu_sc as plsc

MIN_KEPT_PER_BATCH = 100000

B = 16
H = 512
W = 512
N = B * H * W
NT = 16
KROW = 256
ROWS_PER_CHUNK = 8
CHUNK_W = ROWS_PER_CHUNK * W
NCHUNK = KROW // ROWS_PER_CHUNK
NBUCKET = 32768
HROW = NBUCKET // 128
SLICE_ROWS = HROW // NT


def _tc_keys_body(l_ref, g_ref, o_ref):
    x = l_ref[...]
    t = g_ref[...]
    loss = jnp.maximum(x, 0.0) - x * t + jnp.log1p(jnp.exp(-jnp.abs(x)))
    k = jnp.right_shift(lax.bitcast_convert_type(loss, jnp.int32), 16)
    a = k[:, : H // 2, :]
    b = k[:, H // 2:, :]
    o_ref[...] = a | lax.shift_left(b, 16)


def _tc_keys(logits3, gts3):
    return pl.pallas_call(
        _tc_keys_body,
        grid=(4,),
        in_specs=[
            pl.BlockSpec((B // 4, H, W), lambda i: (i, 0, 0)),
            pl.BlockSpec((B // 4, H, W), lambda i: (i, 0, 0)),
        ],
        out_specs=pl.BlockSpec((B // 4, KROW, W), lambda i: (i, 0, 0)),
        out_shape=jax.ShapeDtypeStruct((B, KROW, W), jnp.int32),
    )(logits3, gts3)


def _extract(vec, i):
    lanes = lax.iota(jnp.int32, 16)
    return jnp.sum(jnp.where(lanes == i, vec, 0.0))


def _mid_value(bucket_base, lanes):
    key = lax.shift_left(bucket_base + lanes, 16) | 0x8000
    return plsc.bitcast(key, jnp.float32)


def _sc_select(keys3, kept):
    kept_f = float(kept)

    def body(keys_hbm, out_hbm, bufa, bufb, cnt, scn_c, idx,
             rbuf, tbuf, sema, semb, merged_c, totals):
        s = lax.axis_index("s")
        wid = s
        ones = jnp.full((16,), 1.0, jnp.float32)
        zeros16 = jnp.zeros((16,), jnp.float32)
        lanes = lax.iota(jnp.int32, 16)

        def chunk_src(ci):
            return keys_hbm.at[s, pl.ds(ci * ROWS_PER_CHUNK, ROWS_PER_CHUNK), :]

        def start(ci, buf, sem):
            pltpu.make_async_copy(chunk_src(ci), buf, sem).start()

        def wait(ci, buf, sem):
            pltpu.make_async_copy(chunk_src(ci), buf, sem).wait()

        @plsc.parallel_loop(0, HROW, unroll=8)
        def _(r):
            for c in range(8):
                cnt[r, pl.ds(c * 16, 16)] = zeros16

        def fill_idx(h, _):
            def fv(i, _):
                idx[h, pl.ds(i * 16, 16)] = (
                    lax.iota(jnp.int32, 16) + h * 128 + i * 16)
                return 0
            lax.fori_loop(0, 8, fv, 0)
            return 0
        lax.fori_loop(0, HROW // 128, fill_idx, 0)

        @pl.when(wid == 0)
        def _():
            pltpu.sync_copy(cnt, merged_c)

        def process(buf):
            @plsc.parallel_loop(0, CHUNK_W // 16, unroll=8)
            def _(i):
                r = jnp.right_shift(i, 5)
                c = i & 31
                v = buf[r, pl.ds(c * 16, 16)]
                lo = v & 0xFFFF
                hi = lax.shift_right_logical(v, 16)
                plsc.addupdate_scatter(
                    cnt, [jnp.right_shift(lo, 7), lo & 127], ones)
                plsc.addupdate_scatter(
                    cnt, [jnp.right_shift(hi, 7), hi & 127], ones)

        start(0, bufa, sema)

        def pair(p, _):
            c0 = 2 * p
            wait(c0, bufa, sema)
            start(c0 + 1, bufb, semb)
            process(bufa)
            wait(c0 + 1, bufb, semb)

            @pl.when(c0 + 2 < NCHUNK)
            def _():
                start(c0 + 2, bufa, sema)

            process(bufb)
            return 0

        lax.fori_loop(0, NCHUNK // 2, pair, 0)

        plsc.subcore_barrier()

        for h in range(HROW // 128):
            pltpu.sync_copy(cnt.at[pl.ds(h * 128, 128), :],
                            merged_c.at[idx.at[h]], add=True)
        plsc.subcore_barrier()

        pltpu.sync_copy(merged_c.at[pl.ds(s * SLICE_ROWS, SLICE_ROWS), :],
                        scn_c)

        def tot(i, carry):
            vc, vs = carry
            r = jnp.right_shift(i, 3)
            c = i & 7
            cv = scn_c[r, pl.ds(c * 16, 16)]
            base = (s * SLICE_ROWS + r) * 128 + c * 16
            return (vc + cv, vs + cv * _mid_value(base, lanes))
        vc, vs = lax.fori_loop(0, SLICE_ROWS * 8, tot, (zeros16, zeros16))
        tc_ = jnp.sum(vc)
        ts_ = jnp.sum(vs)
        rbuf[...] = (jnp.where(lanes == 0, tc_, 0.0)
                     + jnp.where(lanes == 1, ts_, 0.0))
        pltpu.sync_copy(rbuf, totals.at[s])
        plsc.subcore_barrier()

        @pl.when(wid == 0)
        def _():
            pltpu.sync_copy(totals, tbuf)

            def coarse(t, carry):
                cum_c, cum_s, sstar, base_c, base_s = carry
                tt = NT - 1 - t
                rv = tbuf[tt]
                tcv = rv[0]
                tsv = rv[1]
                hit = jnp.logical_and(cum_c + tcv >= kept_f, sstar < 0)
                sstar = jnp.where(hit, tt, sstar)
                base_c = jnp.where(hit, cum_c, base_c)
                base_s = jnp.where(hit, cum_s, base_s)
                return (cum_c + tcv, cum_s + tsv, sstar, base_c, base_s)

            _, _, sstar, base_c, base_s = lax.fori_loop(
                0, NT, coarse,
                (0.0, 0.0, jnp.int32(-1), 0.0, 0.0))

            pltpu.sync_copy(
                merged_c.at[pl.ds(sstar * SLICE_ROWS, SLICE_ROWS), :], scn_c)

            def fine(j, carry):
                (cum_c, cum_s, found, cnt_ab, sum_ab, mstar) = carry
                r = SLICE_ROWS - 1 - jnp.right_shift(j, 3)
                cj = 7 - (j & 7)
                vcv = scn_c[r, pl.ds(cj * 16, 16)]
                base = (sstar * SLICE_ROWS + r) * 128 + cj * 16
                midv = _mid_value(base, lanes)
                vsv = vcv * midv
                rc = lax.rev(vcv, (0,))
                rs = lax.rev(vsv, (0,))
                rm = lax.rev(midv, (0,))
                cc = plsc.cumsum(rc)
                cs = plsc.cumsum(rs)
                sfx = cum_c + cc
                msk = sfx >= kept_f
                ntrue = plsc.all_reduce_population_count(msk)[0]
                i0 = 16 - ntrue
                cc_i = _extract(cc, i0)
                cs_i = _extract(cs, i0)
                rc_i = _extract(rc, i0)
                rs_i = _extract(rs, i0)
                rm_i = _extract(rm, i0)
                use = jnp.logical_and(ntrue > 0, jnp.logical_not(found))
                cnt_ab = jnp.where(use, cum_c + cc_i - rc_i, cnt_ab)
                sum_ab = jnp.where(use, cum_s + cs_i - rs_i, sum_ab)
                mstar = jnp.where(use, rm_i, mstar)
                found = jnp.logical_or(found, ntrue > 0)
                return (cum_c + cc[15], cum_s + cs[15], found,
                        cnt_ab, sum_ab, mstar)

            (_, _, _, cnt_ab, sum_ab, mstar) = lax.fori_loop(
                0, SLICE_ROWS * 8, fine,
                (base_c, base_s, jnp.bool_(False), 0.0, 0.0, 0.0))

            resid = kept_f - cnt_ab
            ans = jnp.full((16,), (sum_ab + resid * mstar) * (1.0 / kept_f),
                           jnp.float32)
            rbuf[...] = ans
            pltpu.sync_copy(rbuf, out_hbm)

    mesh = plsc.VectorSubcoreMesh(
        core_axis_name="c", subcore_axis_name="s", num_cores=1)
    f = pl.kernel(
        body,
        out_type=jax.ShapeDtypeStruct((16,), jnp.float32),
        mesh=mesh,
        compiler_params=pltpu.CompilerParams(needs_layout_passes=False),
        scratch_types=[
            pltpu.VMEM((ROWS_PER_CHUNK, W), jnp.int32),
            pltpu.VMEM((ROWS_PER_CHUNK, W), jnp.int32),
            pltpu.VMEM((HROW, 128), jnp.float32),
            pltpu.VMEM((SLICE_ROWS, 128), jnp.float32),
            pltpu.VMEM((HROW // 128, 128), jnp.int32),
            pltpu.VMEM((16,), jnp.float32),
            pltpu.VMEM((NT, 16), jnp.float32),
            pltpu.SemaphoreType.DMA,
            pltpu.SemaphoreType.DMA,
            pltpu.VMEM_SHARED((HROW, 128), jnp.float32),
            pltpu.VMEM_SHARED((NT, 16), jnp.float32),
        ],
    )
    return f(keys3)


@jax.jit
def kernel(logits, gts):
    kept = MIN_KEPT_PER_BATCH * gts.shape[0]
    l3 = logits.reshape(B, H, W)
    g3 = gts.reshape(B, H, W)
    keys = _tc_keys(l3, g3)
    out = _sc_select(keys, kept)
    return out[0]
